# Initial kernel scaffold; baseline (speedup 1.0000x reference)
import jax
import jax.numpy as jnp
from jax.experimental import pallas as pl


def kernel(x, expert_indices, expert_weights, gate_proj, up_proj, down_proj):
    raise NotImplementedError("write your pallas kernel here")


# speedup vs baseline: 1.2540x; 1.2540x over previous
import functools

import jax
import jax.numpy as jnp
from jax import lax
from jax.experimental import pallas as pl
from jax.experimental.pallas import tpu as pltpu
from jax.experimental.pallas import tpu_sc as plsc

T = 256
FB = 1024


def _routing_metadata(expert_indices, expert_weights, E, T, G_MAX):
    P = expert_indices.size
    e = expert_indices.reshape(P).astype(jnp.int32)
    w = expert_weights.reshape(P)
    topk = expert_indices.shape[-1]
    tok = (jnp.arange(P, dtype=jnp.int32) // topk).astype(jnp.int32)

    onehot = (e[:, None] == jnp.arange(E, dtype=jnp.int32)[None, :]).astype(jnp.int32)
    counts = jnp.sum(onehot, axis=0)
    ranks = jnp.cumsum(onehot, axis=0) - onehot
    rank = jnp.take_along_axis(ranks, e[:, None], axis=1)[:, 0]

    tiles_per_e = (counts + T - 1) // T
    tile_start = jnp.concatenate(
        [jnp.zeros((1,), jnp.int32), jnp.cumsum(tiles_per_e).astype(jnp.int32)])
    padded_start = tile_start[:-1] * T
    pos = padded_start[e] + rank

    P_MAX = G_MAX * T
    row_token = jnp.zeros((P_MAX,), jnp.int32).at[pos].set(tok)
    row_weight = jnp.zeros((P_MAX,), jnp.float32).at[pos].set(w)

    g_ids = jnp.arange(G_MAX, dtype=jnp.int32)
    e_of_g = jnp.sum(
        (g_ids[:, None] >= tile_start[1:][None, :]).astype(jnp.int32), axis=1)
    num_tiles = tile_start[E]
    tile_valid = (g_ids < num_tiles).astype(jnp.int32)
    e_clamped = jnp.minimum(e_of_g, E - 1)
    e_last = e_clamped[jnp.maximum(num_tiles - 1, 0)]
    tile_expert = jnp.where(tile_valid == 1, e_clamped, e_last)
    return row_token, row_weight, pos, tile_expert, tile_valid


def _make_sc_gather(P_MAX, H):
    info = plsc.get_sparse_core_info()
    NW = info.num_cores * info.num_subcores
    b_per_w = P_MAX // NW
    chunks = []
    off = 0
    while off < b_per_w:
        c = min(64, b_per_w - off)
        chunks.append((off, c))
        off += c
    mesh = plsc.VectorSubcoreMesh(core_axis_name="c", subcore_axis_name="s")

    @functools.partial(
        pl.kernel,
        out_type=jax.ShapeDtypeStruct((P_MAX, H), jnp.float32),
        mesh=mesh,
        scratch_types=[
            pltpu.VMEM((b_per_w,), jnp.int32),
            pltpu.VMEM((64, H), jnp.float32),
            pltpu.SemaphoreType.DMA,
        ],
    )
    def gather_k(x_hbm, idx_hbm, out_hbm, idx_v, rows_v, sem):
        wid = lax.axis_index("s") * info.num_cores + lax.axis_index("c")
        base = wid * b_per_w
        pltpu.sync_copy(idx_hbm.at[pl.ds(base, b_per_w)], idx_v)
        for off, c in chunks:
            pltpu.async_copy(
                x_hbm.at[idx_v.at[pl.ds(off, c)]], rows_v.at[pl.ds(0, c)], sem
            ).wait()
            pltpu.sync_copy(rows_v.at[pl.ds(0, c)],
                            out_hbm.at[pl.ds(base + off, c)])

    return gather_k


def _make_sc_combine(S, H, P_MAX):
    info = plsc.get_sparse_core_info()
    NW = info.num_cores * info.num_subcores
    t_per_w = S // NW
    CH = 32
    n_ch = t_per_w // CH
    L = info.num_lanes
    mesh = plsc.VectorSubcoreMesh(core_axis_name="c", subcore_axis_name="s")

    @functools.partial(
        pl.kernel,
        out_type=jax.ShapeDtypeStruct((S, H), jnp.float32),
        mesh=mesh,
        scratch_types=[
            pltpu.VMEM((CH,), jnp.int32),
            pltpu.VMEM((CH,), jnp.int32),
            pltpu.VMEM((CH, H), jnp.float32),
            pltpu.VMEM((CH, H), jnp.float32),
            pltpu.SemaphoreType.DMA,
            pltpu.SemaphoreType.DMA,
        ],
    )
    def combine_k(yg_hbm, pos0_hbm, pos1_hbm, out_hbm,
                  i0_v, i1_v, a_v, b_v, sem0, sem1):
        wid = lax.axis_index("s") * info.num_cores + lax.axis_index("c")
        base = wid * t_per_w
        for c in range(n_ch):
            cbase = base + c * CH
            pltpu.sync_copy(pos0_hbm.at[pl.ds(cbase, CH)], i0_v)
            pltpu.sync_copy(pos1_hbm.at[pl.ds(cbase, CH)], i1_v)
            cp0 = pltpu.async_copy(yg_hbm.at[i0_v], a_v, sem0)
            cp1 = pltpu.async_copy(yg_hbm.at[i1_v], b_v, sem1)
            cp0.wait()
            cp1.wait()

            def add_row(r, _):
                def add_vec(v, _):
                    sl = pl.ds(v * L, L)
                    a_v[r, sl] = a_v[r, sl] + b_v[r, sl]
                    return 0
                return lax.fori_loop(0, H // L, add_vec, 0)

            lax.fori_loop(0, CH, add_row, 0)
            pltpu.sync_copy(a_v, out_hbm.at[pl.ds(cbase, CH)])

    return combine_k


def _ffn_body(te_ref, tv_ref, xg_ref, gw_ref, uw_ref, dw_ref, w_ref, out_ref,
              *, n_ff_blocks):
    g = pl.program_id(0)
    j = pl.program_id(1)

    @pl.when(tv_ref[g] == 1)
    def _():
        x = xg_ref[...]
        gate = lax.dot_general(
            x, gw_ref[0], (((1,), (1,)), ((), ())),
            preferred_element_type=jnp.float32)
        up = lax.dot_general(
            x, uw_ref[0], (((1,), (1,)), ((), ())),
            preferred_element_type=jnp.float32)
        inter = (gate * jax.nn.sigmoid(gate)) * up
        part = lax.dot_general(
            inter, dw_ref[0], (((1,), (1,)), ((), ())),
            preferred_element_type=jnp.float32)

        @pl.when(j == 0)
        def _():
            out_ref[...] = part

        @pl.when(j > 0)
        def _():
            out_ref[...] = out_ref[...] + part

        @pl.when(j == n_ff_blocks - 1)
        def _():
            out_ref[...] = out_ref[...] * w_ref[...]


def kernel(x, expert_indices, expert_weights, gate_proj, up_proj, down_proj):
    b, s, h = x.shape
    E, FF, _ = gate_proj.shape
    topk = expert_indices.shape[-1]
    P = b * s * topk
    G_MAX = P // T + (E - 1)
    P_MAX = G_MAX * T
    J = FF // FB

    x_flat = x.reshape(b * s, h)
    row_token, row_weight, pos, tile_expert, tile_valid = _routing_metadata(
        expert_indices, expert_weights, E, T, G_MAX)

    xg = _make_sc_gather(P_MAX, h)(x_flat, row_token)

    grid_spec = pltpu.PrefetchScalarGridSpec(
        num_scalar_prefetch=2,
        grid=(G_MAX, J),
        in_specs=[
            pl.BlockSpec((T, h), lambda g, j, te, tv: (g, 0)),
            pl.BlockSpec((1, FB, h), lambda g, j, te, tv: (te[g], j, 0)),
            pl.BlockSpec((1, FB, h), lambda g, j, te, tv: (te[g], j, 0)),
            pl.BlockSpec((1, h, FB), lambda g, j, te, tv: (te[g], 0, j)),
            pl.BlockSpec((T, 1), lambda g, j, te, tv: (g, 0)),
        ],
        out_specs=pl.BlockSpec((T, h), lambda g, j, te, tv: (g, 0)),
    )
    yg = pl.pallas_call(
        functools.partial(_ffn_body, n_ff_blocks=J),
        grid_spec=grid_spec,
        out_shape=jax.ShapeDtypeStruct((P_MAX, h), jnp.float32),
        compiler_params=pltpu.CompilerParams(
            dimension_semantics=("arbitrary", "arbitrary")),
    )(tile_expert, tile_valid, xg, gate_proj, up_proj, down_proj,
      row_weight.reshape(P_MAX, 1))

    pos2 = pos.reshape(b * s, topk)
    out = _make_sc_combine(b * s, h, P_MAX)(
        yg, pos2[:, 0].astype(jnp.int32), pos2[:, 1].astype(jnp.int32))
    return out.reshape(b, s, h)
